# baseline (speedup 1.0000x reference)
import jax
import jax.numpy as jnp
from jax.experimental import pallas as pl
from jax.experimental.pallas import tpu as pltpu

_RES = 32
_F = 256


def _pos_embed_body(row_hbm, col_hbm, out_hbm, rowbuf, rightbuf, sems):
    cp_row = pltpu.make_async_copy(row_hbm, rowbuf, sems.at[_RES])
    cp_row.start()

    for g in range(_RES):
        pltpu.make_async_copy(
            col_hbm,
            out_hbm.at[pl.ds(g * _RES, _RES), pl.ds(0, _F)],
            sems.at[g],
        ).start()

    cp_row.wait()
    for r in range(_RES):
        rightbuf[pl.ds(r * _RES, _RES), :] = jnp.broadcast_to(
            rowbuf[pl.ds(r, 1), :], (_RES, _F))
    cp_right = pltpu.make_async_copy(
        rightbuf, out_hbm.at[:, pl.ds(_F, _F)], sems.at[_RES + 1])
    cp_right.start()

    for g in range(_RES):
        pltpu.make_async_copy(
            col_hbm,
            out_hbm.at[pl.ds(g * _RES, _RES), pl.ds(0, _F)],
            sems.at[g],
        ).wait()
    cp_right.wait()


@jax.jit
def _pos_embed(row_embed, col_embed):
    return pl.pallas_call(
        _pos_embed_body,
        in_specs=[
            pl.BlockSpec(memory_space=pl.ANY),
            pl.BlockSpec(memory_space=pl.ANY),
        ],
        out_specs=pl.BlockSpec(memory_space=pl.ANY),
        out_shape=jax.ShapeDtypeStruct((_RES * _RES, 2 * _F), jnp.float32),
        scratch_shapes=[
            pltpu.VMEM((_RES, _F), jnp.float32),
            pltpu.VMEM((_RES * _RES, _F), jnp.float32),
            pltpu.SemaphoreType.DMA((_RES + 2,)),
        ],
    )(row_embed, col_embed)


def kernel(row_embed, col_embed):
    pos = _pos_embed(row_embed, col_embed)
    return pos[None, :, :]


# speedup vs baseline: 14.0142x; 14.0142x over previous
import jax
import jax.numpy as jnp
from jax.experimental import pallas as pl
from jax.experimental.pallas import tpu as pltpu

_RES = 32
_F = 256
_NCHUNK = 16
_GPC = _RES // _NCHUNK


def _pos_embed_body(row_ref, col_ref, out_hbm, scratch, sems):
    col = col_ref[...]
    rows_per_chunk = _GPC * _RES
    for c in range(_NCHUNK):
        for i in range(_GPC):
            r = c * _GPC + i
            scratch[pl.ds(r * _RES, _RES), 0:_F] = col
            scratch[pl.ds(r * _RES, _RES), _F:2 * _F] = jnp.broadcast_to(
                row_ref[pl.ds(r, 1), :], (_RES, _F))
        pltpu.make_async_copy(
            scratch.at[pl.ds(c * rows_per_chunk, rows_per_chunk)],
            out_hbm.at[pl.ds(c * rows_per_chunk, rows_per_chunk)],
            sems.at[c],
        ).start()
    for c in range(_NCHUNK):
        pltpu.make_async_copy(
            scratch.at[pl.ds(c * rows_per_chunk, rows_per_chunk)],
            out_hbm.at[pl.ds(c * rows_per_chunk, rows_per_chunk)],
            sems.at[c],
        ).wait()


@jax.jit
def _pos_embed(row_embed, col_embed):
    return pl.pallas_call(
        _pos_embed_body,
        in_specs=[
            pl.BlockSpec(memory_space=pltpu.VMEM),
            pl.BlockSpec(memory_space=pltpu.VMEM),
        ],
        out_specs=pl.BlockSpec(memory_space=pl.ANY),
        out_shape=jax.ShapeDtypeStruct((_RES * _RES, 2 * _F), jnp.float32),
        scratch_shapes=[
            pltpu.VMEM((_RES * _RES, 2 * _F), jnp.float32),
            pltpu.SemaphoreType.DMA((_NCHUNK,)),
        ],
    )(row_embed, col_embed)


def kernel(row_embed, col_embed):
    pos = _pos_embed(row_embed, col_embed)
    return pos[None, :, :]


# speedup vs baseline: 14.4425x; 1.0306x over previous
import jax
import jax.numpy as jnp
from jax.experimental import pallas as pl
from jax.experimental.pallas import tpu as pltpu

_RES = 32
_F = 256
_NCHUNK = 4
_GPC = _RES // _NCHUNK


def _pos_embed_body(row_ref, col_ref, out_hbm, scratch, sems):
    col = col_ref[...]
    rows_per_chunk = _GPC * _RES
    for c in range(_NCHUNK):
        for i in range(_GPC):
            r = c * _GPC + i
            scratch[pl.ds(r * _RES, _RES), 0:_F] = col
            scratch[pl.ds(r * _RES, _RES), _F:2 * _F] = jnp.broadcast_to(
                row_ref[pl.ds(r, 1), :], (_RES, _F))
        pltpu.make_async_copy(
            scratch.at[pl.ds(c * rows_per_chunk, rows_per_chunk)],
            out_hbm.at[pl.ds(c * rows_per_chunk, rows_per_chunk)],
            sems.at[c],
        ).start()
    for c in range(_NCHUNK):
        pltpu.make_async_copy(
            scratch.at[pl.ds(c * rows_per_chunk, rows_per_chunk)],
            out_hbm.at[pl.ds(c * rows_per_chunk, rows_per_chunk)],
            sems.at[c],
        ).wait()


@jax.jit
def _pos_embed(row_embed, col_embed):
    return pl.pallas_call(
        _pos_embed_body,
        in_specs=[
            pl.BlockSpec(memory_space=pltpu.VMEM),
            pl.BlockSpec(memory_space=pltpu.VMEM),
        ],
        out_specs=pl.BlockSpec(memory_space=pl.ANY),
        out_shape=jax.ShapeDtypeStruct((_RES * _RES, 2 * _F), jnp.float32),
        scratch_shapes=[
            pltpu.VMEM((_RES * _RES, 2 * _F), jnp.float32),
            pltpu.SemaphoreType.DMA((_NCHUNK,)),
        ],
    )(row_embed, col_embed)


def kernel(row_embed, col_embed):
    pos = _pos_embed(row_embed, col_embed)
    return pos[None, :, :]
